# baseline (device time: 68844 ns/iter reference)
import jax
import jax.numpy as jnp
from jax import lax
from jax.experimental import pallas as pl
from jax.experimental.pallas import tpu as pltpu

N_DEV = 32
NG = N_DEV // 2
B, SQ, DM = 2, 512, 768
HL, DH = 8, 64
HD = HL * DH
ROWS = B * SQ
CHUNK = SQ // NG
BLK = 64
BF = jnp.bfloat16
F32 = jnp.float32


def kernel(x, Wq, K_ext, V_ext, Wo):
    me_out = lax.axis_index("i")
    wq_l = lax.dynamic_slice(Wq, (0, me_out * HD), (DM, HD))
    wo_l = lax.dynamic_slice(Wo, (me_out * HD, 0), (HD, DM))

    def body(x_ref, wq_ref, k_ref, v_ref, wo_ref, out_ref,
             acc_ref, ctx_ref, p1_buf, rs_buf, gh_buf, oh_buf,
             s1_send, s1_recv, rs_send, rs_recv,
             ag_send, ag_recv, f_send, f_recv):
        me = lax.axis_index("i")
        parity = lax.rem(me, 2)
        partner = me + 1 - 2 * parity
        g = me // 2
        half_off = parity * SQ
        my_chunk = half_off + g * CHUNK

        s1 = pltpu.make_async_remote_copy(
            src_ref=acc_ref.at[pl.ds((1 - parity) * SQ, SQ)],
            dst_ref=p1_buf,
            send_sem=s1_send, recv_sem=s1_recv,
            device_id=(partner,), device_id_type=pl.DeviceIdType.MESH,
        )

        HQ = SQ // 2
        qb = lax.broadcasted_iota(jnp.int32, (HQ, SQ), 0) // BLK
        kb = lax.broadcasted_iota(jnp.int32, (HQ, SQ), 1) // BLK
        mask_lo = (kb[:, :HQ] <= qb[:, :HQ])
        mask_hi = (kb <= (qb + HQ // BLK))
        wq16 = wq_ref[...].astype(BF)
        for phase in range(B):
            b = (1 - parity) if phase == 0 else parity
            q = jnp.dot(x_ref[b].astype(BF), wq16,
                        preferred_element_type=F32).astype(BF)
            for h in range(HL):
                qh = q[:, h * DH:(h + 1) * DH]
                kh = k_ref[b, :, h, :].astype(BF)
                vh = v_ref[b, :, h, :].astype(BF)
                s_lo = lax.dot_general(
                    qh[:HQ], kh[:HQ], (((1,), (1,)), ((), ())),
                    preferred_element_type=F32)
                e_lo = jnp.exp(jnp.where(mask_lo, s_lo * 0.125, -1e9))
                w_lo = (e_lo / jnp.sum(e_lo, axis=1, keepdims=True)).astype(BF)
                ctx_ref[pl.ds(0, HQ), h * DH:(h + 1) * DH] = jnp.dot(
                    w_lo, vh[:HQ], preferred_element_type=F32).astype(BF)
                s_hi = lax.dot_general(
                    qh[HQ:], kh, (((1,), (1,)), ((), ())),
                    preferred_element_type=F32)
                e_hi = jnp.exp(jnp.where(mask_hi, s_hi * 0.125, -1e9))
                w_hi = (e_hi / jnp.sum(e_hi, axis=1, keepdims=True)).astype(BF)
                ctx_ref[pl.ds(HQ, HQ), h * DH:(h + 1) * DH] = jnp.dot(
                    w_hi, vh, preferred_element_type=F32).astype(BF)
            acc_ref[pl.ds(b * SQ, SQ), :] = jnp.dot(
                ctx_ref[...], wo_ref[...].astype(BF),
                preferred_element_type=F32).astype(BF)
            if phase == 0:
                s1.start()

        s1.wait_recv()
        acc_ref[pl.ds(half_off, SQ), :] = (
            acc_ref[pl.ds(half_off, SQ), :].astype(F32)
            + p1_buf[...].astype(F32)).astype(BF)

        rs_descs = []
        for dg in range(1, NG):
            tg = lax.rem(g + dg, NG)
            desc = pltpu.make_async_remote_copy(
                src_ref=acc_ref.at[pl.ds(half_off + tg * CHUNK, CHUNK)],
                dst_ref=rs_buf.at[dg - 1],
                send_sem=rs_send.at[dg - 1],
                recv_sem=rs_recv.at[dg - 1],
                device_id=(2 * tg + parity,),
                device_id_type=pl.DeviceIdType.MESH,
            )
            desc.start()
            rs_descs.append(desc)
        for desc in rs_descs:
            desc.wait_recv()
        chunk = acc_ref[pl.ds(my_chunk, CHUNK), :].astype(F32)
        for dg in range(1, NG):
            chunk = chunk + rs_buf[dg - 1].astype(F32)
        gh_buf[pl.ds(g * CHUNK, CHUNK), :] = chunk.astype(BF)

        ag_descs = []
        for dg in range(1, NG):
            tg = lax.rem(g + dg, NG)
            desc = pltpu.make_async_remote_copy(
                src_ref=gh_buf.at[pl.ds(g * CHUNK, CHUNK)],
                dst_ref=gh_buf.at[pl.ds(g * CHUNK, CHUNK)],
                send_sem=ag_send.at[dg - 1],
                recv_sem=ag_recv.at[dg - 1],
                device_id=(2 * tg + parity,),
                device_id_type=pl.DeviceIdType.MESH,
            )
            desc.start()
            ag_descs.append(desc)

        f_descs = []
        for dg in range(NG):
            r = lax.rem(g - dg + NG, NG) * CHUNK
            f_descs.append(pltpu.make_async_remote_copy(
                src_ref=gh_buf.at[pl.ds(r, CHUNK)],
                dst_ref=oh_buf.at[pl.ds(r, CHUNK)],
                send_sem=f_send.at[dg],
                recv_sem=f_recv.at[dg],
                device_id=(partner,), device_id_type=pl.DeviceIdType.MESH,
            ))
        f_descs[0].start()
        for dg in range(1, NG):
            ag_descs[dg - 1].wait_recv()
            f_descs[dg].start()

        out_ref[pl.ds(half_off, SQ), :] = gh_buf[...].astype(F32)
        for desc in f_descs:
            desc.wait_recv()
        out_ref[pl.ds((1 - parity) * SQ, SQ), :] = oh_buf[...].astype(F32)

        s1.wait_send()
        for desc in rs_descs:
            desc.wait_send()
        for desc in ag_descs:
            desc.wait_send()
        for desc in f_descs:
            desc.wait_send()

    out = pl.pallas_call(
        body,
        out_shape=jax.ShapeDtypeStruct((ROWS, DM), F32),
        in_specs=[pl.BlockSpec(memory_space=pltpu.VMEM)] * 5,
        out_specs=pl.BlockSpec(memory_space=pltpu.VMEM),
        scratch_shapes=[
            pltpu.VMEM((ROWS, DM), BF),
            pltpu.VMEM((SQ, HD), BF),
            pltpu.VMEM((SQ, DM), BF),
            pltpu.VMEM((NG - 1, CHUNK, DM), BF),
            pltpu.VMEM((SQ, DM), BF),
            pltpu.VMEM((SQ, DM), BF),
            pltpu.SemaphoreType.DMA,
            pltpu.SemaphoreType.DMA,
            pltpu.SemaphoreType.DMA((NG - 1,)),
            pltpu.SemaphoreType.DMA((NG - 1,)),
            pltpu.SemaphoreType.DMA((NG - 1,)),
            pltpu.SemaphoreType.DMA((NG - 1,)),
            pltpu.SemaphoreType.DMA((NG,)),
            pltpu.SemaphoreType.DMA((NG,)),
        ],
    )(x, wq_l, K_ext, V_ext, wo_l)
    return out.reshape(B, SQ, DM)


# device time: 68717 ns/iter; 1.0018x vs baseline; 1.0018x over previous
import jax
import jax.numpy as jnp
from jax import lax
from jax.experimental import pallas as pl
from jax.experimental.pallas import tpu as pltpu

N_DEV = 32
NG = N_DEV // 2
B, SQ, DM = 2, 512, 768
HL, DH = 8, 64
HD = HL * DH
ROWS = B * SQ
CHUNK = SQ // NG
BLK = 64
BF = jnp.bfloat16
F32 = jnp.float32


def kernel(x, Wq, K_ext, V_ext, Wo):
    me_out = lax.axis_index("i")
    wq_l = lax.dynamic_slice(Wq, (0, me_out * HD), (DM, HD)).astype(BF)
    wo_l = lax.dynamic_slice(Wo, (me_out * HD, 0), (HD, DM)).astype(BF)
    x16 = x.astype(BF)
    k16 = K_ext.astype(BF)
    v16 = V_ext.astype(BF)

    def body(x_ref, wq_ref, k_ref, v_ref, wo_ref, out_ref,
             acc_ref, ctx_ref, p1_buf, rs_buf, gh_buf, oh_buf,
             s1_send, s1_recv, rs_send, rs_recv,
             ag_send, ag_recv, f_send, f_recv):
        me = lax.axis_index("i")
        parity = lax.rem(me, 2)
        partner = me + 1 - 2 * parity
        g = me // 2
        half_off = parity * SQ
        my_chunk = half_off + g * CHUNK

        s1 = pltpu.make_async_remote_copy(
            src_ref=acc_ref.at[pl.ds((1 - parity) * SQ, SQ)],
            dst_ref=p1_buf,
            send_sem=s1_send, recv_sem=s1_recv,
            device_id=(partner,), device_id_type=pl.DeviceIdType.MESH,
        )

        HQ = SQ // 2
        qb = lax.broadcasted_iota(jnp.int32, (HQ, SQ), 0) // BLK
        kb = lax.broadcasted_iota(jnp.int32, (HQ, SQ), 1) // BLK
        mask_lo = (kb[:, :HQ] <= qb[:, :HQ])
        mask_hi = (kb <= (qb + HQ // BLK))
        for phase in range(B):
            b = (1 - parity) if phase == 0 else parity
            q = jnp.dot(x_ref[b], wq_ref[...],
                        preferred_element_type=F32).astype(BF)
            for h in range(HL):
                qh = q[:, h * DH:(h + 1) * DH]
                kh = k_ref[b, :, h, :]
                vh = v_ref[b, :, h, :]
                s_lo = lax.dot_general(
                    qh[:HQ], kh[:HQ], (((1,), (1,)), ((), ())),
                    preferred_element_type=F32)
                e_lo = jnp.exp(jnp.where(mask_lo, s_lo * 0.125, -1e9))
                w_lo = (e_lo / jnp.sum(e_lo, axis=1, keepdims=True)).astype(BF)
                ctx_ref[pl.ds(0, HQ), h * DH:(h + 1) * DH] = jnp.dot(
                    w_lo, vh[:HQ], preferred_element_type=F32).astype(BF)
                s_hi = lax.dot_general(
                    qh[HQ:], kh, (((1,), (1,)), ((), ())),
                    preferred_element_type=F32)
                e_hi = jnp.exp(jnp.where(mask_hi, s_hi * 0.125, -1e9))
                w_hi = (e_hi / jnp.sum(e_hi, axis=1, keepdims=True)).astype(BF)
                ctx_ref[pl.ds(HQ, HQ), h * DH:(h + 1) * DH] = jnp.dot(
                    w_hi, vh, preferred_element_type=F32).astype(BF)
            acc_ref[pl.ds(b * SQ, SQ), :] = jnp.dot(
                ctx_ref[...], wo_ref[...],
                preferred_element_type=F32).astype(BF)
            if phase == 0:
                s1.start()

        s1.wait_recv()
        acc_ref[pl.ds(half_off, SQ), :] = (
            acc_ref[pl.ds(half_off, SQ), :].astype(F32)
            + p1_buf[...].astype(F32)).astype(BF)

        rs_descs = []
        for dg in range(1, NG):
            tg = lax.rem(g + dg, NG)
            desc = pltpu.make_async_remote_copy(
                src_ref=acc_ref.at[pl.ds(half_off + tg * CHUNK, CHUNK)],
                dst_ref=rs_buf.at[dg - 1],
                send_sem=rs_send.at[dg - 1],
                recv_sem=rs_recv.at[dg - 1],
                device_id=(2 * tg + parity,),
                device_id_type=pl.DeviceIdType.MESH,
            )
            desc.start()
            rs_descs.append(desc)
        for desc in rs_descs:
            desc.wait_recv()
        chunk = acc_ref[pl.ds(my_chunk, CHUNK), :].astype(F32)
        for dg in range(1, NG):
            chunk = chunk + rs_buf[dg - 1].astype(F32)
        gh_buf[pl.ds(g * CHUNK, CHUNK), :] = chunk.astype(BF)

        ag_descs = []
        for dg in range(1, NG):
            tg = lax.rem(g + dg, NG)
            desc = pltpu.make_async_remote_copy(
                src_ref=gh_buf.at[pl.ds(g * CHUNK, CHUNK)],
                dst_ref=gh_buf.at[pl.ds(g * CHUNK, CHUNK)],
                send_sem=ag_send.at[dg - 1],
                recv_sem=ag_recv.at[dg - 1],
                device_id=(2 * tg + parity,),
                device_id_type=pl.DeviceIdType.MESH,
            )
            desc.start()
            ag_descs.append(desc)

        f_descs = []
        for dg in range(NG):
            r = lax.rem(g - dg + NG, NG) * CHUNK
            f_descs.append(pltpu.make_async_remote_copy(
                src_ref=gh_buf.at[pl.ds(r, CHUNK)],
                dst_ref=oh_buf.at[pl.ds(r, CHUNK)],
                send_sem=f_send.at[dg],
                recv_sem=f_recv.at[dg],
                device_id=(partner,), device_id_type=pl.DeviceIdType.MESH,
            ))
        f_descs[0].start()
        for dg in range(1, NG):
            ag_descs[dg - 1].wait_recv()
            f_descs[dg].start()

        out_ref[pl.ds(half_off, SQ), :] = gh_buf[...].astype(F32)
        for desc in f_descs:
            desc.wait_recv()
        out_ref[pl.ds((1 - parity) * SQ, SQ), :] = oh_buf[...].astype(F32)

        s1.wait_send()
        for desc in rs_descs:
            desc.wait_send()
        for desc in ag_descs:
            desc.wait_send()
        for desc in f_descs:
            desc.wait_send()

    out = pl.pallas_call(
        body,
        out_shape=jax.ShapeDtypeStruct((ROWS, DM), F32),
        in_specs=[pl.BlockSpec(memory_space=pltpu.VMEM)] * 5,
        out_specs=pl.BlockSpec(memory_space=pltpu.VMEM),
        scratch_shapes=[
            pltpu.VMEM((ROWS, DM), BF),
            pltpu.VMEM((SQ, HD), BF),
            pltpu.VMEM((SQ, DM), BF),
            pltpu.VMEM((NG - 1, CHUNK, DM), BF),
            pltpu.VMEM((SQ, DM), BF),
            pltpu.VMEM((SQ, DM), BF),
            pltpu.SemaphoreType.DMA,
            pltpu.SemaphoreType.DMA,
            pltpu.SemaphoreType.DMA((NG - 1,)),
            pltpu.SemaphoreType.DMA((NG - 1,)),
            pltpu.SemaphoreType.DMA((NG - 1,)),
            pltpu.SemaphoreType.DMA((NG - 1,)),
            pltpu.SemaphoreType.DMA((NG,)),
            pltpu.SemaphoreType.DMA((NG,)),
        ],
    )(x16, wq_l, k16, v16, wo_l)
    return out.reshape(B, SQ, DM)


# device time: 68337 ns/iter; 1.0074x vs baseline; 1.0056x over previous
import jax
import jax.numpy as jnp
from jax import lax
from jax.experimental import pallas as pl
from jax.experimental.pallas import tpu as pltpu

N_DEV = 32
NG = N_DEV // 2
B, SQ, DM = 2, 512, 768
HL, DH = 8, 64
HD = HL * DH
ROWS = B * SQ
CHUNK = SQ // NG
BLK = 64
BF = jnp.bfloat16
F32 = jnp.float32


def kernel(x, Wq, K_ext, V_ext, Wo):
    me_out = lax.axis_index("i")
    wq_l = lax.dynamic_slice(Wq, (0, me_out * HD), (DM, HD)).astype(BF)
    wo_l = lax.dynamic_slice(Wo, (me_out * HD, 0), (HD, DM)).astype(BF)
    x16 = x.astype(BF)
    k16 = K_ext.astype(BF)
    v16 = V_ext.astype(BF)

    def body(x_ref, wq_ref, k_ref, v_ref, wo_ref, out_ref,
             acc_ref, ctx_ref, p1_buf, rs_buf, gh_buf, oh_buf,
             s1_send, s1_recv, rs_send, rs_recv,
             ag_send, ag_recv, f_send, f_recv):
        me = lax.axis_index("i")
        parity = lax.rem(me, 2)
        partner = me + 1 - 2 * parity
        g = me // 2
        half_off = parity * SQ
        my_chunk = half_off + g * CHUNK

        s1 = pltpu.make_async_remote_copy(
            src_ref=acc_ref.at[pl.ds((1 - parity) * SQ, SQ)],
            dst_ref=p1_buf,
            send_sem=s1_send, recv_sem=s1_recv,
            device_id=(partner,), device_id_type=pl.DeviceIdType.MESH,
        )

        HQ = SQ // 2
        qb = lax.broadcasted_iota(jnp.int32, (HQ, SQ), 0) // BLK
        kb = lax.broadcasted_iota(jnp.int32, (HQ, SQ), 1) // BLK
        mask_lo = (kb[:, :HQ] <= qb[:, :HQ])
        mask_hi = (kb <= (qb + HQ // BLK))
        for phase in range(B):
            b = (1 - parity) if phase == 0 else parity
            q = jnp.dot(x_ref[b], wq_ref[...],
                        preferred_element_type=F32).astype(BF)
            for h in range(HL):
                qh = q[:, h * DH:(h + 1) * DH]
                kh = k_ref[b, :, h, :]
                vh = v_ref[b, :, h, :]
                s_lo = lax.dot_general(
                    qh[:HQ], kh[:HQ], (((1,), (1,)), ((), ())),
                    preferred_element_type=F32)
                e_lo = jnp.exp(jnp.where(mask_lo, s_lo * 0.125, -1e9))
                w_lo = (e_lo / jnp.sum(e_lo, axis=1, keepdims=True)).astype(BF)
                ctx_ref[pl.ds(0, HQ), h * DH:(h + 1) * DH] = jnp.dot(
                    w_lo, vh[:HQ], preferred_element_type=F32).astype(BF)
                s_hi = lax.dot_general(
                    qh[HQ:], kh, (((1,), (1,)), ((), ())),
                    preferred_element_type=F32)
                e_hi = jnp.exp(jnp.where(mask_hi, s_hi * 0.125, -1e9))
                w_hi = (e_hi / jnp.sum(e_hi, axis=1, keepdims=True)).astype(BF)
                ctx_ref[pl.ds(HQ, HQ), h * DH:(h + 1) * DH] = jnp.dot(
                    w_hi, vh, preferred_element_type=F32).astype(BF)
            acc_ref[pl.ds(b * SQ, SQ), :] = jnp.dot(
                ctx_ref[...], wo_ref[...],
                preferred_element_type=F32).astype(BF)
            if phase == 0:
                s1.start()

        s1.wait_recv()
        rs_descs = []
        for dg in range(1, NG):
            tg = lax.rem(g + dg, NG)
            acc_ref[pl.ds(half_off + tg * CHUNK, CHUNK), :] = (
                acc_ref[pl.ds(half_off + tg * CHUNK, CHUNK), :].astype(F32)
                + p1_buf[pl.ds(tg * CHUNK, CHUNK), :].astype(F32)).astype(BF)
            desc = pltpu.make_async_remote_copy(
                src_ref=acc_ref.at[pl.ds(half_off + tg * CHUNK, CHUNK)],
                dst_ref=rs_buf.at[dg - 1],
                send_sem=rs_send.at[dg - 1],
                recv_sem=rs_recv.at[dg - 1],
                device_id=(2 * tg + parity,),
                device_id_type=pl.DeviceIdType.MESH,
            )
            desc.start()
            rs_descs.append(desc)
        chunk = (acc_ref[pl.ds(my_chunk, CHUNK), :].astype(F32)
                 + p1_buf[pl.ds(g * CHUNK, CHUNK), :].astype(F32))
        for dg in range(1, NG):
            rs_descs[dg - 1].wait_recv()
            chunk = chunk + rs_buf[dg - 1].astype(F32)
        gh_buf[pl.ds(g * CHUNK, CHUNK), :] = chunk.astype(BF)

        ag_descs = []
        for dg in range(1, NG):
            tg = lax.rem(g + dg, NG)
            desc = pltpu.make_async_remote_copy(
                src_ref=gh_buf.at[pl.ds(g * CHUNK, CHUNK)],
                dst_ref=gh_buf.at[pl.ds(g * CHUNK, CHUNK)],
                send_sem=ag_send.at[dg - 1],
                recv_sem=ag_recv.at[dg - 1],
                device_id=(2 * tg + parity,),
                device_id_type=pl.DeviceIdType.MESH,
            )
            desc.start()
            ag_descs.append(desc)

        f_descs = []
        for dg in range(NG):
            r = lax.rem(g - dg + NG, NG) * CHUNK
            f_descs.append(pltpu.make_async_remote_copy(
                src_ref=gh_buf.at[pl.ds(r, CHUNK)],
                dst_ref=oh_buf.at[pl.ds(r, CHUNK)],
                send_sem=f_send.at[dg],
                recv_sem=f_recv.at[dg],
                device_id=(partner,), device_id_type=pl.DeviceIdType.MESH,
            ))
        f_descs[0].start()
        for dg in range(1, NG):
            ag_descs[dg - 1].wait_recv()
            f_descs[dg].start()

        out_ref[pl.ds(half_off, SQ), :] = gh_buf[...].astype(F32)
        oh_off = (1 - parity) * SQ
        for dg in range(NG):
            f_descs[dg].wait_recv()
            r = lax.rem(g - dg + NG, NG) * CHUNK
            out_ref[pl.ds(oh_off + r, CHUNK), :] = (
                oh_buf[pl.ds(r, CHUNK), :].astype(F32))

        s1.wait_send()
        for desc in rs_descs:
            desc.wait_send()
        for desc in ag_descs:
            desc.wait_send()
        for desc in f_descs:
            desc.wait_send()

    out = pl.pallas_call(
        body,
        out_shape=jax.ShapeDtypeStruct((ROWS, DM), F32),
        in_specs=[pl.BlockSpec(memory_space=pltpu.VMEM)] * 5,
        out_specs=pl.BlockSpec(memory_space=pltpu.VMEM),
        scratch_shapes=[
            pltpu.VMEM((ROWS, DM), BF),
            pltpu.VMEM((SQ, HD), BF),
            pltpu.VMEM((SQ, DM), BF),
            pltpu.VMEM((NG - 1, CHUNK, DM), BF),
            pltpu.VMEM((SQ, DM), BF),
            pltpu.VMEM((SQ, DM), BF),
            pltpu.SemaphoreType.DMA,
            pltpu.SemaphoreType.DMA,
            pltpu.SemaphoreType.DMA((NG - 1,)),
            pltpu.SemaphoreType.DMA((NG - 1,)),
            pltpu.SemaphoreType.DMA((NG - 1,)),
            pltpu.SemaphoreType.DMA((NG - 1,)),
            pltpu.SemaphoreType.DMA((NG,)),
            pltpu.SemaphoreType.DMA((NG,)),
        ],
    )(x16, wq_l, k16, v16, wo_l)
    return out.reshape(B, SQ, DM)
